# baseline (speedup 1.0000x reference)
import functools

import jax
import jax.numpy as jnp
from jax import lax
from jax.experimental import pallas as pl
from jax.experimental.pallas import tpu as pltpu
from jax.experimental.pallas import tpu_sc as plsc

N = 10000
NP = 10240
E = 320000
NHID = 128

NC = 2
NS = 16
NW = NC * NS
CHUNK = 40
SUP = 50
NSUP = E // (NW * CHUNK * SUP)
RPT = NP // NS


def _make_spmm():
    mesh = plsc.VectorSubcoreMesh(core_axis_name="c", subcore_axis_name="s")

    @functools.partial(
        pl.kernel,
        out_type=jax.ShapeDtypeStruct((NC, NP, NHID), jnp.float32),
        mesh=mesh,
        scratch_types=[
            pltpu.VMEM((1, SUP, CHUNK), jnp.int32),
            pltpu.VMEM((1, SUP, CHUNK), jnp.int32),
            pltpu.VMEM((1, SUP, CHUNK), jnp.float32),
            pltpu.VMEM((CHUNK, NHID), jnp.float32),
            pltpu.VMEM((CHUNK, NHID), jnp.float32),
            pltpu.VMEM_SHARED((NP, NHID), jnp.float32),
            pltpu.SemaphoreType.DMA,
            pltpu.SemaphoreType.DMA,
            pltpu.SemaphoreType.DMA,
            pltpu.SemaphoreType.DMA,
            pltpu.SemaphoreType.DMA,
        ],
    )
    def spmm(src_hbm, dst_hbm, w_hbm, x_hbm, zeros_hbm, out_hbm,
             src_v, dst_v, w_v, rows_a, rows_b, acc,
             sem_ga, sem_gb, sem_sa, sem_sb, sem_st):
        i32 = jnp.int32
        c = lax.axis_index("c").astype(i32)
        s = lax.axis_index("s").astype(i32)
        wid = c * i32(NS) + s

        pltpu.sync_copy(zeros_hbm, acc.at[pl.ds(s * i32(RPT), RPT)])
        plsc.subcore_barrier()

        def stage_start(u, par):
            pltpu.async_copy(src_hbm.at[wid, u], src_v.at[par], sem_st)
            pltpu.async_copy(dst_hbm.at[wid, u], dst_v.at[par], sem_st)
            pltpu.async_copy(w_hbm.at[wid, u], w_v.at[par], sem_st)

        def stage_wait(u, par):
            pltpu.make_async_copy(src_hbm.at[wid, u], src_v.at[par],
                                  sem_st).wait()
            pltpu.make_async_copy(dst_hbm.at[wid, u], dst_v.at[par],
                                  sem_st).wait()
            pltpu.make_async_copy(w_hbm.at[wid, u], w_v.at[par],
                                  sem_st).wait()

        def gather_start(par, j, buf, sem):
            pltpu.async_copy(x_hbm.at[src_v.at[par].at[j]], buf, sem)

        def gather_wait(par, j, buf, sem):
            pltpu.make_async_copy(x_hbm.at[src_v.at[par].at[j]], buf,
                                  sem).wait()

        def scatter_start(par, j, buf, sem):
            pltpu.async_copy(buf, acc.at[dst_v.at[par].at[j]], sem, add=True)

        def scatter_wait(par, j, buf, sem):
            pltpu.make_async_copy(buf, acc.at[dst_v.at[par].at[j]],
                                  sem).wait()

        def scale(par, j, buf):
            def scale16(g, carry2):
                w16 = w_v[par, j, pl.ds(g * i32(16), 16)]
                for r in range(16):
                    wr = w16[r]
                    row = g * i32(16) + i32(r)
                    for cj in range(NHID // 16):
                        sl = pl.ds(cj * 16, 16)
                        buf[row, sl] = buf[row, sl] * wr
                return carry2

            lax.fori_loop(i32(0), i32(CHUNK // 16), scale16, i32(0))

        def superchunk(u, carry0):
            par = i32(0)
            stage_start(u, par)
            stage_wait(u, par)

            gather_start(par, i32(0), rows_a, sem_ga)

            def pair(p, carry):
                a = p * i32(2)

                @pl.when(p > i32(0))
                def _():
                    scatter_wait(par, a - i32(1), rows_b, sem_sb)

                gather_start(par, a + i32(1), rows_b, sem_gb)
                gather_wait(par, a, rows_a, sem_ga)
                scale(par, a, rows_a)
                scatter_start(par, a, rows_a, sem_sa)
                gather_wait(par, a + i32(1), rows_b, sem_gb)
                scale(par, a + i32(1), rows_b)
                scatter_wait(par, a, rows_a, sem_sa)

                @pl.when(p + i32(1) < i32(SUP // 2))
                def _():
                    gather_start(par, a + i32(2), rows_a, sem_ga)

                scatter_start(par, a + i32(1), rows_b, sem_sb)
                return carry

            lax.fori_loop(i32(0), i32(SUP // 2), pair, i32(0))
            scatter_wait(par, i32(SUP - 1), rows_b, sem_sb)
            return carry0

        lax.fori_loop(i32(0), i32(NSUP), superchunk, i32(0))
        plsc.subcore_barrier()
        pltpu.sync_copy(acc.at[pl.ds(s * i32(RPT), RPT)],
                        out_hbm.at[c, pl.ds(s * i32(RPT), RPT)])

    return spmm


_spmm = _make_spmm()


BLK = 1024


def _dot3(a, b):
    f32 = jnp.float32
    bf = jnp.bfloat16
    ah = a.astype(bf)
    al = (a - ah.astype(f32)).astype(bf)
    bh = b.astype(bf)
    bl = (b - bh.astype(f32)).astype(bf)

    def d(u, v):
        return jnp.dot(u, v, preferred_element_type=f32)

    return d(ah, bh) + (d(ah, bl) + d(al, bh))


def _tc_layer_body(p_ref, w_ref, lw1_ref, lb1_ref, lw2_ref, lb2_ref,
                   lw3_ref, lb3_ref, sacc_ref, y_ref, s_ref):
    h = p_ref[0] + p_ref[1]
    r = jnp.maximum(h, 0.0)
    nrm = jnp.sqrt(jnp.sum(r * r, axis=1, keepdims=True))
    x = r / jnp.maximum(nrm, 1e-12)
    y_ref[...] = _dot3(x, w_ref[...])
    m = jnp.maximum(_dot3(x, lw1_ref[...]) + lb1_ref[...], 0.0)
    m = jnp.maximum(_dot3(m, lw2_ref[...]) + lb2_ref[...], 0.0)
    s_ref[...] = sacc_ref[...] + _dot3(m, lw3_ref[...]) + lb3_ref[...]


def _tc_tail_body(p_ref, lw1_ref, lb1_ref, lw2_ref, lb2_ref,
                  lw3_ref, lb3_ref, sacc_ref, mul_ref, s_ref):
    h = p_ref[0] + p_ref[1]
    x = jnp.maximum(h, 0.0)
    m = jnp.maximum(_dot3(x, lw1_ref[...]) + lb1_ref[...], 0.0)
    m = jnp.maximum(_dot3(m, lw2_ref[...]) + lb2_ref[...], 0.0)
    s = sacc_ref[...] + _dot3(m, lw3_ref[...]) + lb3_ref[...]
    s_ref[...] = s * mul_ref[...]


def _full(shape):
    return pl.BlockSpec(shape, lambda *_: tuple(jnp.int32(0) for _ in shape))


def _tc_layer(p, w, lw1, lb1, lw2, lb2, lw3, lb3, sacc):
    return pl.pallas_call(
        _tc_layer_body,
        grid=(NP // BLK,),
        in_specs=[
            pl.BlockSpec((NC, BLK, NHID),
                         lambda i: (jnp.int32(0), i, jnp.int32(0))),
            _full((NHID, NHID)),
            _full((NHID, 2 * NHID)), _full((1, 2 * NHID)),
            _full((2 * NHID, 2 * NHID)), _full((1, 2 * NHID)),
            _full((2 * NHID, 1)), _full((1, 1)),
            pl.BlockSpec((BLK, 1), lambda i: (i, jnp.int32(0))),
        ],
        out_specs=[
            pl.BlockSpec((BLK, NHID), lambda i: (i, jnp.int32(0))),
            pl.BlockSpec((BLK, 1), lambda i: (i, jnp.int32(0))),
        ],
        out_shape=[
            jax.ShapeDtypeStruct((NP, NHID), jnp.float32),
            jax.ShapeDtypeStruct((NP, 1), jnp.float32),
        ],
    )(p, w, lw1, lb1, lw2, lb2, lw3, lb3, sacc)


def _tc_tail(p, lw1, lb1, lw2, lb2, lw3, lb3, sacc, mul):
    return pl.pallas_call(
        _tc_tail_body,
        grid=(NP // BLK,),
        in_specs=[
            pl.BlockSpec((NC, BLK, NHID),
                         lambda i: (jnp.int32(0), i, jnp.int32(0))),
            _full((NHID, 2 * NHID)), _full((1, 2 * NHID)),
            _full((2 * NHID, 2 * NHID)), _full((1, 2 * NHID)),
            _full((2 * NHID, 1)), _full((1, 1)),
            pl.BlockSpec((BLK, 1), lambda i: (i, jnp.int32(0))),
            pl.BlockSpec((BLK, 1), lambda i: (i, jnp.int32(0))),
        ],
        out_specs=pl.BlockSpec((BLK, 1), lambda i: (i, jnp.int32(0))),
        out_shape=jax.ShapeDtypeStruct((NP, 1), jnp.float32),
    )(p, lw1, lb1, lw2, lb2, lw3, lb3, sacc, mul)


def kernel(edge_index1, edge_weight1, edge_index2, edge_weight2,
           W1, W2, W3, W4, lw1, lb1, lw2, lb2, lw3, lb3):
    out_dtype = jnp.result_type(edge_weight1.dtype, W1.dtype, lw1.dtype,
                                lb1.dtype, lw3.dtype)
    f32 = jnp.float32
    W1, W2, W3, W4 = (a.astype(f32) for a in (W1, W2, W3, W4))
    lw1, lw2, lw3 = (a.astype(f32) for a in (lw1, lw2, lw3))
    lb1, lb2, lb3 = (a.astype(f32) for a in (lb1, lb2, lb3))
    zeros = jnp.zeros((RPT, NHID), jnp.float32)
    lb1r = lb1.reshape(1, 2 * NHID)
    lb2r = lb2.reshape(1, 2 * NHID)
    lb3r = lb3.reshape(1, 1)
    s0 = jnp.zeros((NP, 1), jnp.float32)
    ones = jnp.ones((NP, 1), jnp.float32)

    def edges(ei, ew):
        i = ei.astype(jnp.int32).reshape(2, NW, NSUP, SUP, CHUNK)
        return i[1], i[0], ew.astype(f32).reshape(NW, NSUP, SUP, CHUNK)

    eb1 = edges(edge_index1, edge_weight1)
    eb2 = edges(edge_index2, edge_weight2)
    p1 = _spmm(*eb1, W1, zeros)
    p2 = _spmm(*eb2, W1, zeros)
    y1, sc1 = _tc_layer(p1, W2, lw1, lb1r, lw2, lb2r, lw3, lb3r, s0)
    y2, sc2 = _tc_layer(p2, W2, lw1, lb1r, lw2, lb2r, lw3, lb3r, s0)
    for Wn in (W3, W4):
        p1 = _spmm(*eb1, y1, zeros)
        p2 = _spmm(*eb2, y2, zeros)
        y1, sc1 = _tc_layer(p1, Wn, lw1, lb1r, lw2, lb2r, lw3, lb3r, sc1)
        y2, sc2 = _tc_layer(p2, Wn, lw1, lb1r, lw2, lb2r, lw3, lb3r, sc2)
    p1 = _spmm(*eb1, y1, zeros)
    p2 = _spmm(*eb2, y2, zeros)
    s1 = _tc_tail(p1, lw1, lb1r, lw2, lb2r, lw3, lb3r, sc1, ones)
    out = _tc_tail(p2, lw1, lb1r, lw2, lb2r, lw3, lb3r, sc2, s1)
    return out[:N].astype(out_dtype)


# speedup vs baseline: 1.1921x; 1.1921x over previous
import functools

import jax
import jax.numpy as jnp
from jax import lax
from jax.experimental import pallas as pl
from jax.experimental.pallas import tpu as pltpu
from jax.experimental.pallas import tpu_sc as plsc

N = 10000
NP = 10240
E = 320000
NHID = 128

NC = 2
NS = 16
NW = NC * NS
CHUNK = 80
SUP = 25
NSUP = E // (NW * CHUNK * SUP)
RPT = NP // NS


def _make_spmm():
    mesh = plsc.VectorSubcoreMesh(core_axis_name="c", subcore_axis_name="s")

    @functools.partial(
        pl.kernel,
        out_type=jax.ShapeDtypeStruct((NC, NP, NHID), jnp.float32),
        mesh=mesh,
        scratch_types=[
            pltpu.VMEM((1, SUP, CHUNK), jnp.int32),
            pltpu.VMEM((1, SUP, CHUNK), jnp.int32),
            pltpu.VMEM((1, SUP, CHUNK), jnp.float32),

            pltpu.VMEM((CHUNK, NHID), jnp.float32),
            pltpu.VMEM((CHUNK, NHID), jnp.float32),
            pltpu.VMEM_SHARED((NP, NHID), jnp.float32),
            pltpu.SemaphoreType.DMA,
            pltpu.SemaphoreType.DMA,
            pltpu.SemaphoreType.DMA,
            pltpu.SemaphoreType.DMA,
            pltpu.SemaphoreType.DMA,
        ],
    )
    def spmm(src_hbm, dst_hbm, w_hbm, x_hbm, zeros_hbm, out_hbm,
             src_v, dst_v, w_v, rows_a, rows_b, acc,
             sem_ga, sem_gb, sem_sa, sem_sb, sem_st):
        i32 = jnp.int32
        c = lax.axis_index("c").astype(i32)
        s = lax.axis_index("s").astype(i32)
        wid = c * i32(NS) + s

        pltpu.sync_copy(zeros_hbm, acc.at[pl.ds(s * i32(RPT), RPT)])
        plsc.subcore_barrier()

        def stage_start(u, par):
            pltpu.async_copy(src_hbm.at[wid, u], src_v.at[par], sem_st)
            pltpu.async_copy(dst_hbm.at[wid, u], dst_v.at[par], sem_st)
            pltpu.async_copy(w_hbm.at[wid, u], w_v.at[par], sem_st)

        def stage_wait(u, par):
            pltpu.make_async_copy(src_hbm.at[wid, u], src_v.at[par],
                                  sem_st).wait()
            pltpu.make_async_copy(dst_hbm.at[wid, u], dst_v.at[par],
                                  sem_st).wait()
            pltpu.make_async_copy(w_hbm.at[wid, u], w_v.at[par],
                                  sem_st).wait()

        def gather_start(par, j, buf, sem):
            pltpu.async_copy(x_hbm.at[src_v.at[par].at[j]], buf, sem)

        def gather_wait(par, j, buf, sem):
            pltpu.make_async_copy(x_hbm.at[src_v.at[par].at[j]], buf,
                                  sem).wait()

        def scatter_start(par, j, buf, sem):
            pltpu.async_copy(buf, acc.at[dst_v.at[par].at[j]], sem, add=True)

        def scatter_wait(par, j, buf, sem):
            pltpu.make_async_copy(buf, acc.at[dst_v.at[par].at[j]],
                                  sem).wait()

        def scale(par, j, buf):
            def scale16(g, carry2):
                w16 = w_v[par, j, pl.ds(g * i32(16), 16)]
                for r in range(16):
                    wr = w16[r]
                    row = g * i32(16) + i32(r)
                    for cj in range(NHID // 16):
                        sl = pl.ds(cj * 16, 16)
                        buf[row, sl] = buf[row, sl] * wr
                return carry2

            lax.fori_loop(i32(0), i32(CHUNK // 16), scale16, i32(0))

        def superchunk(u, carry0):
            par = i32(0)
            stage_start(u, par)
            stage_wait(u, par)

            gather_start(par, i32(0), rows_a, sem_ga)

            def pair(p, carry):
                a = p * i32(2)

                @pl.when(p > i32(0))
                def _():
                    scatter_wait(par, a - i32(1), rows_b, sem_sb)

                gather_start(par, a + i32(1), rows_b, sem_gb)
                gather_wait(par, a, rows_a, sem_ga)
                scale(par, a, rows_a)
                scatter_start(par, a, rows_a, sem_sa)
                gather_wait(par, a + i32(1), rows_b, sem_gb)
                scale(par, a + i32(1), rows_b)
                scatter_wait(par, a, rows_a, sem_sa)

                gather_start(par, a + i32(2), rows_a, sem_ga)
                scatter_start(par, a + i32(1), rows_b, sem_sb)
                return carry

            lax.fori_loop(i32(0), i32(SUP // 2), pair, i32(0))
            last = i32(SUP - 1)
            gather_wait(par, last, rows_a, sem_ga)
            scale(par, last, rows_a)
            scatter_wait(par, last - i32(1), rows_b, sem_sb)
            scatter_start(par, last, rows_a, sem_sa)
            scatter_wait(par, last, rows_a, sem_sa)
            return carry0

        lax.fori_loop(i32(0), i32(NSUP), superchunk, i32(0))
        plsc.subcore_barrier()
        pltpu.sync_copy(acc.at[pl.ds(s * i32(RPT), RPT)],
                        out_hbm.at[c, pl.ds(s * i32(RPT), RPT)])

    return spmm


_spmm = _make_spmm()


BLK = 1024


def _dot3(a, b):
    f32 = jnp.float32
    bf = jnp.bfloat16
    ah = a.astype(bf)
    al = (a - ah.astype(f32)).astype(bf)
    bh = b.astype(bf)
    bl = (b - bh.astype(f32)).astype(bf)

    def d(u, v):
        return jnp.dot(u, v, preferred_element_type=f32)

    return d(ah, bh) + (d(ah, bl) + d(al, bh))


def _tc_layer_body(p_ref, w_ref, lw1_ref, lb1_ref, lw2_ref, lb2_ref,
                   lw3_ref, lb3_ref, sacc_ref, y_ref, s_ref):
    h = p_ref[0] + p_ref[1]
    r = jnp.maximum(h, 0.0)
    nrm = jnp.sqrt(jnp.sum(r * r, axis=1, keepdims=True))
    x = r / jnp.maximum(nrm, 1e-12)
    y_ref[...] = _dot3(x, w_ref[...])
    m = jnp.maximum(_dot3(x, lw1_ref[...]) + lb1_ref[...], 0.0)
    m = jnp.maximum(_dot3(m, lw2_ref[...]) + lb2_ref[...], 0.0)
    s_ref[...] = sacc_ref[...] + _dot3(m, lw3_ref[...]) + lb3_ref[...]


def _tc_tail_body(p_ref, lw1_ref, lb1_ref, lw2_ref, lb2_ref,
                  lw3_ref, lb3_ref, sacc_ref, mul_ref, s_ref):
    h = p_ref[0] + p_ref[1]
    x = jnp.maximum(h, 0.0)
    m = jnp.maximum(_dot3(x, lw1_ref[...]) + lb1_ref[...], 0.0)
    m = jnp.maximum(_dot3(m, lw2_ref[...]) + lb2_ref[...], 0.0)
    s = sacc_ref[...] + _dot3(m, lw3_ref[...]) + lb3_ref[...]
    s_ref[...] = s * mul_ref[...]


def _full(shape):
    return pl.BlockSpec(shape, lambda *_: tuple(jnp.int32(0) for _ in shape))


def _tc_layer(p, w, lw1, lb1, lw2, lb2, lw3, lb3, sacc):
    return pl.pallas_call(
        _tc_layer_body,
        grid=(NP // BLK,),
        in_specs=[
            pl.BlockSpec((NC, BLK, NHID),
                         lambda i: (jnp.int32(0), i, jnp.int32(0))),
            _full((NHID, NHID)),
            _full((NHID, 2 * NHID)), _full((1, 2 * NHID)),
            _full((2 * NHID, 2 * NHID)), _full((1, 2 * NHID)),
            _full((2 * NHID, 1)), _full((1, 1)),
            pl.BlockSpec((BLK, 1), lambda i: (i, jnp.int32(0))),
        ],
        out_specs=[
            pl.BlockSpec((BLK, NHID), lambda i: (i, jnp.int32(0))),
            pl.BlockSpec((BLK, 1), lambda i: (i, jnp.int32(0))),
        ],
        out_shape=[
            jax.ShapeDtypeStruct((NP, NHID), jnp.float32),
            jax.ShapeDtypeStruct((NP, 1), jnp.float32),
        ],
    )(p, w, lw1, lb1, lw2, lb2, lw3, lb3, sacc)


def _tc_tail(p, lw1, lb1, lw2, lb2, lw3, lb3, sacc, mul):
    return pl.pallas_call(
        _tc_tail_body,
        grid=(NP // BLK,),
        in_specs=[
            pl.BlockSpec((NC, BLK, NHID),
                         lambda i: (jnp.int32(0), i, jnp.int32(0))),
            _full((NHID, 2 * NHID)), _full((1, 2 * NHID)),
            _full((2 * NHID, 2 * NHID)), _full((1, 2 * NHID)),
            _full((2 * NHID, 1)), _full((1, 1)),
            pl.BlockSpec((BLK, 1), lambda i: (i, jnp.int32(0))),
            pl.BlockSpec((BLK, 1), lambda i: (i, jnp.int32(0))),
        ],
        out_specs=pl.BlockSpec((BLK, 1), lambda i: (i, jnp.int32(0))),
        out_shape=jax.ShapeDtypeStruct((NP, 1), jnp.float32),
    )(p, lw1, lb1, lw2, lb2, lw3, lb3, sacc, mul)


def kernel(edge_index1, edge_weight1, edge_index2, edge_weight2,
           W1, W2, W3, W4, lw1, lb1, lw2, lb2, lw3, lb3):
    out_dtype = jnp.result_type(edge_weight1.dtype, W1.dtype, lw1.dtype,
                                lb1.dtype, lw3.dtype)
    f32 = jnp.float32
    W1, W2, W3, W4 = (a.astype(f32) for a in (W1, W2, W3, W4))
    lw1, lw2, lw3 = (a.astype(f32) for a in (lw1, lw2, lw3))
    lb1, lb2, lb3 = (a.astype(f32) for a in (lb1, lb2, lb3))
    zeros = jnp.zeros((RPT, NHID), jnp.float32)
    lb1r = lb1.reshape(1, 2 * NHID)
    lb2r = lb2.reshape(1, 2 * NHID)
    lb3r = lb3.reshape(1, 1)
    s0 = jnp.zeros((NP, 1), jnp.float32)
    ones = jnp.ones((NP, 1), jnp.float32)

    def edges(ei, ew):
        i = ei.astype(jnp.int32).reshape(2, NW, NSUP, SUP, CHUNK)
        return i[1], i[0], ew.astype(f32).reshape(NW, NSUP, SUP, CHUNK)

    eb1 = edges(edge_index1, edge_weight1)
    eb2 = edges(edge_index2, edge_weight2)
    p1 = _spmm(*eb1, W1, zeros)
    p2 = _spmm(*eb2, W1, zeros)
    y1, sc1 = _tc_layer(p1, W2, lw1, lb1r, lw2, lb2r, lw3, lb3r, s0)
    y2, sc2 = _tc_layer(p2, W2, lw1, lb1r, lw2, lb2r, lw3, lb3r, s0)
    for Wn in (W3, W4):
        p1 = _spmm(*eb1, y1, zeros)
        p2 = _spmm(*eb2, y2, zeros)
        y1, sc1 = _tc_layer(p1, Wn, lw1, lb1r, lw2, lb2r, lw3, lb3r, sc1)
        y2, sc2 = _tc_layer(p2, Wn, lw1, lb1r, lw2, lb2r, lw3, lb3r, sc2)
    p1 = _spmm(*eb1, y1, zeros)
    p2 = _spmm(*eb2, y2, zeros)
    s1 = _tc_tail(p1, lw1, lb1r, lw2, lb2r, lw3, lb3r, sc1, ones)
    out = _tc_tail(p2, lw1, lb1r, lw2, lb2r, lw3, lb3r, sc2, s1)
    return out[:N].astype(out_dtype)
